# baseline (speedup 1.0000x reference)
import jax
import jax.numpy as jnp
from jax import lax
from jax.experimental import pallas as pl
from jax.experimental.pallas import tpu as pltpu
from jax.experimental.pallas import tpu_sc as plsc

N_ATOMS = 10000
N_BONDS = 160000
HIDDEN = 256
FDIM = 151
MAX_NB = 16
N_MOLS = 500
ATOMS_PER_MOL = 20
FEAT_DIM = 200
FFN_HID = 1024
NUM_TASKS = 12

NW = 32
ATOMS_PAD = 10240
ATOMS_PER_W = ATOMS_PAD // NW
CA = 4
NBUF = 6
FIRE_AHEAD = 5
NACC = 3
CH0 = 160
SL0 = 168


def _gather_sum_body(bond_hbm, idx_hbm, out_hbm, idx_all,
                     rows0, rows1, rows2, rows3, rows4, rows5,
                     acc0, acc1, acc2,
                     sem0, sem1, sem2, sem3, sem4, sem5,
                     osem0, osem1, osem2):
    cnum = lax.axis_index("c")
    sid = lax.axis_index("s")
    base_atom = sid * (CH0 * CA)
    nchunks = CH0
    nslots = SL0
    row_bufs = (rows0, rows1, rows2, rows3, rows4, rows5)
    acc_bufs = (acc0, acc1, acc2)
    sems = (sem0, sem1, sem2, sem3, sem4, sem5)
    osems = (osem0, osem1, osem2)

    def idx_slice(slot):
        off = lax.rem(slot, nchunks) * (CA * MAX_NB)
        return idx_all.at[pl.ds(off, CA * MAX_NB)]

    def pair_out_slice(slot):
        p = lax.div(lax.rem(slot, nchunks), 2)
        return out_hbm.at[pl.ds(base_atom + p * (2 * CA), 2 * CA)]

    def fire(slot, b):
        pltpu.async_copy(bond_hbm.at[idx_slice(slot)], row_bufs[b], sems[b])

    def drain(slot, b):
        pltpu.make_async_copy(bond_hbm.at[idx_slice(slot)],
                              row_bufs[b], sems[b]).wait()

    def accumulate(slot, b, wait_out):
        rows_v = row_bufs[b]
        acc_v = acc_bufs[b // 2]
        half = b % 2
        if half == 0 and wait_out:
            pltpu.make_async_copy(acc_v, pair_out_slice(slot - 2 * NACC + 1),
                                  osems[b // 2]).wait()

        def atom_body(a, carry2):
            for cc in range(HIDDEN // 16):
                sl = pl.ds(cc * 16, 16)
                vals = [rows_v[a * MAX_NB + r, sl] for r in range(MAX_NB)]
                while len(vals) > 1:
                    vals = [vals[k] + vals[k + 1]
                            for k in range(0, len(vals), 2)]
                acc_v[half * CA + a, sl] = vals[0]
            return carry2

        lax.fori_loop(0, CA, atom_body, 0, unroll=False)
        if half == 1:
            pltpu.async_copy(acc_v, pair_out_slice(slot), osems[b // 2])

    @pl.when(cnum == 0)
    def _():
        pltpu.sync_copy(
            idx_hbm.at[pl.ds(base_atom * MAX_NB, CH0 * CA * MAX_NB)], idx_all)

        for s0 in range(FIRE_AHEAD):
            fire(s0, s0)

        def ring_pass(i, wait_out):
            for b in range(NBUF):
                slot = i * NBUF + b
                drain(slot, b)
                accumulate(slot, b, wait_out)

                @pl.when(slot + FIRE_AHEAD < nslots)
                def _():
                    fire(slot + FIRE_AHEAD, (b + FIRE_AHEAD) % NBUF)

        ring_pass(0, wait_out=False)

        def ring_body(i, carry):
            ring_pass(i, wait_out=True)
            return carry

        lax.fori_loop(1, nslots // NBUF, ring_body, 0, unroll=False)
        for k in range(NACC):
            pltpu.make_async_copy(acc_bufs[k],
                                  pair_out_slice(nslots - 2 * NACC + 2 * k + 1),
                                  osems[k]).wait()


def _gather_sum(bond_output, idx_flat):
    mesh = plsc.VectorSubcoreMesh(core_axis_name="c", subcore_axis_name="s")
    return pl.kernel(
        _gather_sum_body,
        mesh=mesh,
        out_type=jax.ShapeDtypeStruct((ATOMS_PAD, HIDDEN), jnp.float32),
        scratch_types=[
            pltpu.VMEM((CH0 * CA * MAX_NB,), jnp.int32),
        ] + [pltpu.VMEM((CA * MAX_NB, HIDDEN), jnp.float32)] * NBUF
          + [pltpu.VMEM((2 * CA, HIDDEN), jnp.float32)] * NACC
          + [pltpu.SemaphoreType.DMA] * NBUF
          + [pltpu.SemaphoreType.DMA] * NACC,
    )(bond_output, idx_flat)


BA = 1000
BM = 50
NBLK = N_ATOMS // BA


def _atom_ffn_body(fa_ref, ag_ref, w1a_ref, w1b_ref, b1_ref, w2_ref, b2_ref,
                   g_ref, be_ref, mv_ref):
    h = jnp.dot(fa_ref[...], w1a_ref[...], preferred_element_type=jnp.float32)
    h = h + jnp.dot(ag_ref[...], w1b_ref[...], preferred_element_type=jnp.float32)
    h = jax.nn.relu(h + b1_ref[...])
    o = jnp.dot(h, w2_ref[...], preferred_element_type=jnp.float32) + b2_ref[...]
    mu = jnp.mean(o, axis=-1, keepdims=True)
    xc = o - mu
    var = jnp.mean(xc * xc, axis=-1, keepdims=True)
    ln = xc * lax.rsqrt(var + 1e-5) * g_ref[...] + be_ref[...]
    rows = lax.broadcasted_iota(jnp.int32, (BM, BA), 0)
    cols = lax.broadcasted_iota(jnp.int32, (BM, BA), 1)
    pool = jnp.where(cols // ATOMS_PER_MOL == rows,
                     jnp.float32(1.0 / ATOMS_PER_MOL), jnp.float32(0.0))
    mv_ref[0] = jnp.dot(pool, ln, preferred_element_type=jnp.float32)


def _atom_ffn(f_atoms, aggr, W1a, W1b, b1, W2, b2, ln_scale, ln_bias):
    return pl.pallas_call(
        _atom_ffn_body,
        grid=(NBLK,),
        in_specs=[
            pl.BlockSpec((BA, FDIM), lambda i: (i, 0)),
            pl.BlockSpec((BA, HIDDEN), lambda i: (i, 0)),
            pl.BlockSpec((FDIM, FFN_HID), lambda i: (0, 0)),
            pl.BlockSpec((HIDDEN, FFN_HID), lambda i: (0, 0)),
            pl.BlockSpec((1, FFN_HID), lambda i: (0, 0)),
            pl.BlockSpec((FFN_HID, HIDDEN), lambda i: (0, 0)),
            pl.BlockSpec((1, HIDDEN), lambda i: (0, 0)),
            pl.BlockSpec((1, HIDDEN), lambda i: (0, 0)),
            pl.BlockSpec((1, HIDDEN), lambda i: (0, 0)),
        ],
        out_specs=pl.BlockSpec((1, BM, HIDDEN), lambda i: (i, 0, 0)),
        out_shape=jax.ShapeDtypeStruct((NBLK, BM, HIDDEN), jnp.float32),
    )(f_atoms, aggr, W1a, W1b, b1, W2, b2, ln_scale, ln_bias).reshape(N_MOLS, HIDDEN)


def _mol_ffn_body(mv_ref, fb_ref, wa_ref, wb_ref, b1_ref, w2_ref, b2_ref, out_ref):
    h = jnp.dot(mv_ref[...], wa_ref[...], preferred_element_type=jnp.float32)
    h = h + jnp.dot(fb_ref[...], wb_ref[...], preferred_element_type=jnp.float32)
    h = jax.nn.relu(h + b1_ref[...])
    o = jnp.dot(h, w2_ref[...], preferred_element_type=jnp.float32) + b2_ref[...]
    out_ref[...] = jax.nn.sigmoid(o)


def _mol_ffn(mol_vecs, features_batch, Wf1a, Wf1b, bf1, Wf2, bf2):
    return pl.pallas_call(
        _mol_ffn_body,
        out_shape=jax.ShapeDtypeStruct((N_MOLS, NUM_TASKS), jnp.float32),
    )(mol_vecs, features_batch, Wf1a, Wf1b, bf1, Wf2, bf2)


def kernel(atom_output, bond_output, original_f_atoms, features_batch,
           W1, b1, W2, b2, ln_scale, ln_bias, Wf1, bf1, Wf2, bf2,
           a2b, a_scope):
    idx_flat = jnp.pad(a2b.reshape(-1), (0, (ATOMS_PAD - N_ATOMS) * MAX_NB))
    aggr = _gather_sum(bond_output, idx_flat)
    W1a = W1[:FDIM]
    W1b = W1[FDIM:]
    mol_vecs = _atom_ffn(original_f_atoms, aggr, W1a, W1b,
                         b1.reshape(1, -1), W2, b2.reshape(1, -1),
                         ln_scale.reshape(1, -1), ln_bias.reshape(1, -1))
    Wf1a = Wf1[:HIDDEN]
    Wf1b = Wf1[HIDDEN:]
    return _mol_ffn(mol_vecs, features_batch, Wf1a, Wf1b,
                    bf1.reshape(1, -1), Wf2, bf2.reshape(1, -1))


# speedup vs baseline: 1.2772x; 1.2772x over previous
import jax
import jax.numpy as jnp
from jax import lax
from jax.experimental import pallas as pl
from jax.experimental.pallas import tpu as pltpu
from jax.experimental.pallas import tpu_sc as plsc

N_ATOMS = 10000
N_BONDS = 160000
HIDDEN = 256
FDIM = 151
MAX_NB = 16
N_MOLS = 500
ATOMS_PER_MOL = 20
FEAT_DIM = 200
FFN_HID = 1024
NUM_TASKS = 12

NW = 32
ATOMS_PAD = 10240
ATOMS_PER_W = ATOMS_PAD // NW
CA = 8
NCHUNKS = ATOMS_PER_W // CA


def _gather_sum_body(bond_hbm, idx_hbm, out_hbm,
                     idx0, idx1, rows0, rows1, acc0, acc1, sem0, sem1):
    wid = lax.axis_index("s") * 2 + lax.axis_index("c")
    base_atom = wid * ATOMS_PER_W
    idx_bufs = (idx0, idx1)
    row_bufs = (rows0, rows1)
    acc_bufs = (acc0, acc1)
    sems = (sem0, sem1)

    def fire(c, parity):
        cw = lax.rem(c, NCHUNKS)
        a0 = base_atom + cw * CA
        pltpu.sync_copy(idx_hbm.at[pl.ds(a0 * MAX_NB, CA * MAX_NB)],
                        idx_bufs[parity])
        pltpu.async_copy(bond_hbm.at[idx_bufs[parity]], row_bufs[parity],
                         sems[parity])

    def drain(parity):
        pltpu.make_async_copy(bond_hbm.at[idx_bufs[parity]],
                              row_bufs[parity], sems[parity]).wait()

    def accumulate(c, parity):
        rows_v = row_bufs[parity]
        acc_v = acc_bufs[parity]

        def atom_pair(j, carry2):
            for aa in range(2):
                a = j * 2 + aa
                for cc in range(HIDDEN // 16):
                    sl = pl.ds(cc * 16, 16)
                    vals = [rows_v[a * MAX_NB + r, sl] for r in range(MAX_NB)]
                    while len(vals) > 1:
                        vals = [vals[k] + vals[k + 1]
                                for k in range(0, len(vals), 2)]
                    acc_v[a, sl] = vals[0]
            return carry2

        lax.fori_loop(0, CA // 2, atom_pair, 0, unroll=False)
        pltpu.sync_copy(acc_v, out_hbm.at[pl.ds(base_atom + c * CA, CA)])

    fire(0, 0)

    def pair_body(i, carry):
        c = i * 2
        fire(c + 1, 1)
        drain(0)
        accumulate(c, 0)
        fire(c + 2, 0)
        drain(1)
        accumulate(c + 1, 1)
        return carry

    lax.fori_loop(0, NCHUNKS // 2, pair_body, 0, unroll=False)
    drain(0)


def _gather_sum(bond_output, idx_flat):
    mesh = plsc.VectorSubcoreMesh(core_axis_name="c", subcore_axis_name="s")
    return pl.kernel(
        _gather_sum_body,
        mesh=mesh,
        out_type=jax.ShapeDtypeStruct((ATOMS_PAD, HIDDEN), jnp.float32),
        scratch_types=[
            pltpu.VMEM((CA * MAX_NB,), jnp.int32),
            pltpu.VMEM((CA * MAX_NB,), jnp.int32),
            pltpu.VMEM((CA * MAX_NB, HIDDEN), jnp.float32),
            pltpu.VMEM((CA * MAX_NB, HIDDEN), jnp.float32),
            pltpu.VMEM((CA, HIDDEN), jnp.float32),
            pltpu.VMEM((CA, HIDDEN), jnp.float32),
            pltpu.SemaphoreType.DMA,
            pltpu.SemaphoreType.DMA,
        ],
    )(bond_output, idx_flat)


BA = 1000
BM = 50
NBLK = N_ATOMS // BA


def _atom_ffn_body(fa_ref, ag_ref, w1a_ref, w1b_ref, b1_ref, w2_ref, b2_ref,
                   g_ref, be_ref, mv_ref):
    bf = jnp.bfloat16
    h = jnp.dot(fa_ref[...].astype(bf), w1a_ref[...],
                preferred_element_type=jnp.float32)
    h = h + jnp.dot(ag_ref[...].astype(bf), w1b_ref[...],
                    preferred_element_type=jnp.float32)
    h = jax.nn.relu(h + b1_ref[...])
    o = jnp.dot(h.astype(bf), w2_ref[...],
                preferred_element_type=jnp.float32) + b2_ref[...]
    mu = jnp.mean(o, axis=-1, keepdims=True)
    xc = o - mu
    var = jnp.mean(xc * xc, axis=-1, keepdims=True)
    ln = xc * lax.rsqrt(var + 1e-5) * g_ref[...] + be_ref[...]
    rows = lax.broadcasted_iota(jnp.int32, (BM, BA), 0)
    cols = lax.broadcasted_iota(jnp.int32, (BM, BA), 1)
    pool = jnp.where(cols // ATOMS_PER_MOL == rows,
                     jnp.float32(1.0 / ATOMS_PER_MOL), jnp.float32(0.0))
    mv_ref[0] = jnp.dot(pool, ln, preferred_element_type=jnp.float32)


def _atom_ffn(f_atoms, aggr, W1a, W1b, b1, W2, b2, ln_scale, ln_bias):
    return pl.pallas_call(
        _atom_ffn_body,
        grid=(NBLK,),
        in_specs=[
            pl.BlockSpec((BA, FDIM), lambda i: (i, 0)),
            pl.BlockSpec((BA, HIDDEN), lambda i: (i, 0)),
            pl.BlockSpec((FDIM, FFN_HID), lambda i: (0, 0)),
            pl.BlockSpec((HIDDEN, FFN_HID), lambda i: (0, 0)),
            pl.BlockSpec((1, FFN_HID), lambda i: (0, 0)),
            pl.BlockSpec((FFN_HID, HIDDEN), lambda i: (0, 0)),
            pl.BlockSpec((1, HIDDEN), lambda i: (0, 0)),
            pl.BlockSpec((1, HIDDEN), lambda i: (0, 0)),
            pl.BlockSpec((1, HIDDEN), lambda i: (0, 0)),
        ],
        out_specs=pl.BlockSpec((1, BM, HIDDEN), lambda i: (i, 0, 0)),
        out_shape=jax.ShapeDtypeStruct((NBLK, BM, HIDDEN), jnp.float32),
    )(f_atoms, aggr, W1a, W1b, b1, W2, b2, ln_scale, ln_bias).reshape(N_MOLS, HIDDEN)


def _mol_ffn_body(mv_ref, fb_ref, wa_ref, wb_ref, b1_ref, w2_ref, b2_ref, out_ref):
    bf = jnp.bfloat16
    h = jnp.dot(mv_ref[...].astype(bf), wa_ref[...],
                preferred_element_type=jnp.float32)
    h = h + jnp.dot(fb_ref[...].astype(bf), wb_ref[...],
                    preferred_element_type=jnp.float32)
    h = jax.nn.relu(h + b1_ref[...])
    o = jnp.dot(h.astype(bf), w2_ref[...],
                preferred_element_type=jnp.float32) + b2_ref[...]
    out_ref[...] = jax.nn.sigmoid(o)


def _mol_ffn(mol_vecs, features_batch, Wf1a, Wf1b, bf1, Wf2, bf2):
    return pl.pallas_call(
        _mol_ffn_body,
        out_shape=jax.ShapeDtypeStruct((N_MOLS, NUM_TASKS), jnp.float32),
    )(mol_vecs, features_batch, Wf1a, Wf1b, bf1, Wf2, bf2)


def kernel(atom_output, bond_output, original_f_atoms, features_batch,
           W1, b1, W2, b2, ln_scale, ln_bias, Wf1, bf1, Wf2, bf2,
           a2b, a_scope):
    idx_flat = jnp.pad(a2b.reshape(-1), (0, (ATOMS_PAD - N_ATOMS) * MAX_NB))
    aggr = _gather_sum(bond_output, idx_flat)
    W1a = W1[:FDIM].astype(jnp.bfloat16)
    W1b = W1[FDIM:].astype(jnp.bfloat16)
    mol_vecs = _atom_ffn(original_f_atoms, aggr, W1a, W1b,
                         b1.reshape(1, -1), W2.astype(jnp.bfloat16),
                         b2.reshape(1, -1),
                         ln_scale.reshape(1, -1), ln_bias.reshape(1, -1))
    Wf1a = Wf1[:HIDDEN].astype(jnp.bfloat16)
    Wf1b = Wf1[HIDDEN:].astype(jnp.bfloat16)
    return _mol_ffn(mol_vecs, features_batch, Wf1a, Wf1b,
                    bf1.reshape(1, -1), Wf2.astype(jnp.bfloat16),
                    bf2.reshape(1, -1))
